# Initial kernel scaffold; baseline (speedup 1.0000x reference)
import jax
import jax.numpy as jnp
from jax.experimental import pallas as pl


def kernel(x, edge_index, batch, W0, asrc0, adst0, b0, W1, asrc1, adst1, b1, W2, asrc2, adst2, b2, W_lin, b_lin):
    raise NotImplementedError("write your pallas kernel here")


# speedup vs baseline: 12.8892x; 12.8892x over previous
import jax
import jax.numpy as jnp
from jax import lax
from jax.experimental import pallas as pl
from jax.experimental.pallas import tpu as pltpu
from jax.experimental.pallas import tpu_sc as plsc

N = 10000
E = 320000
F = 128
G = 64

NC = 2
NS = 16
NW = NC * NS
T = 10240
E_PAD = T * NW
C = 128
S = 2048
NSEG = T // S
CPS = S // C
NCH = T // C
FW = F + 16
NPAD = 10240
RPT = NPAD // NS

BN = 1000
NB = N // BN

_SC_PARAMS = pltpu.CompilerParams(needs_layout_passes=False,
                                  use_tc_tiling_on_sc=False)


def _aug(h):
    n = h.shape[0]
    return jnp.concatenate(
        [h, jnp.ones((n, 1), jnp.float32), jnp.zeros((n, FW - F - 1),
                                                     jnp.float32)], axis=1)


def _attn_tail(h, as_w, ad_w, as_ref, ad_ref, ws_ref):
    a_s = jnp.sum(h * as_w, axis=1, keepdims=True)
    a_d = jnp.sum(h * ad_w, axis=1, keepdims=True)
    as_ref[...] = a_s
    ad_ref[...] = a_d
    al = a_s + a_d
    al = jnp.maximum(al, 0.2 * al)
    ws_ref[...] = jnp.exp(al)


def _mm_attn_body(x_ref, W_ref, as_w_ref, ad_w_ref,
                  h_ref, as_ref, ad_ref, ws_ref):
    h = jnp.dot(x_ref[...], W_ref[...], preferred_element_type=jnp.float32,
                 precision=lax.Precision.HIGHEST)
    h_ref[...] = _aug(h)
    _attn_tail(h, as_w_ref[...], ad_w_ref[...], as_ref, ad_ref, ws_ref)


def _layer_in(x, W, asrc, adst):
    return pl.pallas_call(
        _mm_attn_body,
        grid=(NB,),
        in_specs=[
            pl.BlockSpec((BN, F), lambda i: (i, 0)),
            pl.BlockSpec((F, F), lambda i: (0, 0)),
            pl.BlockSpec((1, F), lambda i: (0, 0)),
            pl.BlockSpec((1, F), lambda i: (0, 0)),
        ],
        out_specs=[
            pl.BlockSpec((BN, FW), lambda i: (i, 0)),
            pl.BlockSpec((BN, 1), lambda i: (i, 0)),
            pl.BlockSpec((BN, 1), lambda i: (i, 0)),
            pl.BlockSpec((BN, 1), lambda i: (i, 0)),
        ],
        out_shape=[
            jax.ShapeDtypeStruct((N, FW), jnp.float32),
            jax.ShapeDtypeStruct((N, 1), jnp.float32),
            jax.ShapeDtypeStruct((N, 1), jnp.float32),
            jax.ShapeDtypeStruct((N, 1), jnp.float32),
        ],
    )(x, W, asrc, adst)


def _node_out(acc, ws, haug, b):
    h = haug[:, :F]
    num = acc[0, :, :F] + acc[1, :, :F] + ws * h
    d = acc[0, :, F:F + 1] + acc[1, :, F:F + 1] + ws
    xn = num / d + b
    return jnp.where(xn > 0, xn, jnp.exp(jnp.minimum(xn, 0.0)) - 1.0)


def _combine_body(acc_ref, ws_ref, h_ref, b_ref, W_ref,
                  as_w_ref, ad_w_ref,
                  h2_ref, as_ref, ad_ref, ws2_ref):
    xn = _node_out(acc_ref[...], ws_ref[...], h_ref[...], b_ref[...])
    h2 = jnp.dot(xn, W_ref[...], preferred_element_type=jnp.float32,
                 precision=lax.Precision.HIGHEST)
    h2_ref[...] = _aug(h2)
    _attn_tail(h2, as_w_ref[...], ad_w_ref[...], as_ref, ad_ref, ws2_ref)


def _combine(acc, ws, haug, b, W, asrc, adst):
    return pl.pallas_call(
        _combine_body,
        grid=(NB,),
        in_specs=[
            pl.BlockSpec((2, BN, FW), lambda i: (0, i, 0)),
            pl.BlockSpec((BN, 1), lambda i: (i, 0)),
            pl.BlockSpec((BN, FW), lambda i: (i, 0)),
            pl.BlockSpec((1, F), lambda i: (0, 0)),
            pl.BlockSpec((F, F), lambda i: (0, 0)),
            pl.BlockSpec((1, F), lambda i: (0, 0)),
            pl.BlockSpec((1, F), lambda i: (0, 0)),
        ],
        out_specs=[
            pl.BlockSpec((BN, FW), lambda i: (i, 0)),
            pl.BlockSpec((BN, 1), lambda i: (i, 0)),
            pl.BlockSpec((BN, 1), lambda i: (i, 0)),
            pl.BlockSpec((BN, 1), lambda i: (i, 0)),
        ],
        out_shape=[
            jax.ShapeDtypeStruct((N, FW), jnp.float32),
            jax.ShapeDtypeStruct((N, 1), jnp.float32),
            jax.ShapeDtypeStruct((N, 1), jnp.float32),
            jax.ShapeDtypeStruct((N, 1), jnp.float32),
        ],
    )(acc, ws, haug, b, W, asrc, adst)


def _final_body(acc_ref, ws_ref, h_ref, b_ref, Wl_ref, bl_ref,
                batch_ref, sums_ref, cnts_ref, res_ref):
    i = pl.program_id(0)
    xn = _node_out(acc_ref[...], ws_ref[...], h_ref[...], b_ref[...])
    z = jnp.dot(xn, Wl_ref[...], preferred_element_type=jnp.float32,
                 precision=lax.Precision.HIGHEST)
    gid = lax.broadcasted_iota(jnp.int32, (1, G), 1)
    m = (batch_ref[...] == gid).astype(jnp.float32)
    part = jnp.sum(m * z, axis=0, keepdims=True)
    cnt = jnp.sum(m, axis=0, keepdims=True)

    @pl.when(i == 0)
    def _():
        sums_ref[...] = part
        cnts_ref[...] = cnt

    @pl.when(i > 0)
    def _():
        sums_ref[...] = sums_ref[...] + part
        cnts_ref[...] = cnts_ref[...] + cnt

    @pl.when(i == NB - 1)
    def _():
        res_ref[...] = (sums_ref[...] / jnp.maximum(cnts_ref[...], 1.0)
                        + bl_ref[...])


def _final(acc, ws, haug, b, W_lin, b_lin, batch2d):
    return pl.pallas_call(
        _final_body,
        grid=(NB,),
        in_specs=[
            pl.BlockSpec((2, BN, FW), lambda i: (0, i, 0)),
            pl.BlockSpec((BN, 1), lambda i: (i, 0)),
            pl.BlockSpec((BN, FW), lambda i: (i, 0)),
            pl.BlockSpec((1, F), lambda i: (0, 0)),
            pl.BlockSpec((F, 1), lambda i: (0, 0)),
            pl.BlockSpec((1, 1), lambda i: (0, 0)),
            pl.BlockSpec((BN, 1), lambda i: (i, 0)),
        ],
        out_specs=[
            pl.BlockSpec((1, G), lambda i: (0, 0)),
            pl.BlockSpec((1, G), lambda i: (0, 0)),
            pl.BlockSpec((1, G), lambda i: (0, 0)),
        ],
        out_shape=[
            jax.ShapeDtypeStruct((1, G), jnp.float32),
            jax.ShapeDtypeStruct((1, G), jnp.float32),
            jax.ShapeDtypeStruct((1, G), jnp.float32),
        ],
    )(acc, ws, haug, b, W_lin, b_lin, batch2d)


def _sc_mesh():
    return plsc.VectorSubcoreMesh(core_axis_name="c", subcore_axis_name="s",
                                  num_cores=NC, num_subcores=NS)


def _sc_w_body(as_hbm, ad_hbm, src_hbm, dst_hbm,
               w_hbm,
               src_t, dst_t, asv, adv, w_t):
    c = lax.axis_index("c")
    s = lax.axis_index("s")
    wid = s * NC + c
    lane = lax.iota(jnp.int32, 16)

    pltpu.sync_copy(src_hbm.at[pl.ds(wid * T, T)], src_t)
    pltpu.sync_copy(dst_hbm.at[pl.ds(wid * T, T)], dst_t)
    pltpu.sync_copy(as_hbm, asv)
    pltpu.sync_copy(ad_hbm, adv)

    def _chunk(k, carry):
        base = k * C
        for v in range(C // 16):
            s16 = src_t[pl.ds(base + v * 16, 16)]
            d16 = dst_t[pl.ds(base + v * 16, 16)]
            al = (plsc.load_gather(asv, [s16])
                  + plsc.load_gather(adv, [d16]))
            al = jnp.maximum(al, 0.2 * al)
            w = jnp.exp(al)
            eid = wid * T + base + v * 16 + lane
            w = jnp.where(eid < E, w, 0.0)
            w_t[pl.ds(base + v * 16, 16)] = w
        return carry

    lax.fori_loop(0, NCH, _chunk, 0)
    pltpu.sync_copy(w_t, w_hbm.at[pl.ds(wid * T, T)])


def _sc_msg_body(h_hbm, w_hbm, src_hbm, dst_hbm,
                 acc_hbm,
                 src_g, dst_g, w_g, srcbuf, dstbuf, wbuf,
                 rows, acc_s, sem):
    c = lax.axis_index("c")
    s = lax.axis_index("s")
    wid = s * NC + c
    zero16 = jnp.zeros((16,), jnp.float32)

    def _zr(i, carry):
        for q in range(FW // 16):
            rows[i, pl.ds(q * 16, 16)] = zero16
        return carry

    lax.fori_loop(0, C, _zr, 0)
    for ss in range(NS):
        @pl.when(s == ss)
        def _():
            for j in range(RPT // C):
                pltpu.sync_copy(rows, acc_s.at[pl.ds(ss * RPT + j * C, C)])
    plsc.subcore_barrier()

    def _seg(g, carry):
        off = wid * T + g * S
        pltpu.sync_copy(src_hbm.at[pl.ds(off, S)], src_g)
        pltpu.sync_copy(dst_hbm.at[pl.ds(off, S)], dst_g)
        pltpu.sync_copy(w_hbm.at[pl.ds(off, S)], w_g)

        def _chunk(kk, carry2):
            base = kk * C
            for v in range(C // 16):
                srcbuf[pl.ds(v * 16, 16)] = src_g[pl.ds(base + v * 16, 16)]
                dstbuf[pl.ds(v * 16, 16)] = dst_g[pl.ds(base + v * 16, 16)]
                wbuf[pl.ds(v * 16, 16)] = w_g[pl.ds(base + v * 16, 16)]
            pltpu.async_copy(h_hbm.at[srcbuf], rows, sem).wait()

            def _scale(j, carry3):
                wj = plsc.load_gather(
                    wbuf, [jnp.broadcast_to(j, (16,)).astype(jnp.int32)])
                for q in range(FW // 16):
                    rows[j, pl.ds(q * 16, 16)] = (
                        rows[j, pl.ds(q * 16, 16)] * wj)
                return carry3

            lax.fori_loop(0, C, _scale, 0)
            pltpu.sync_copy(rows, acc_s.at[dstbuf], add=True)
            return carry2

        lax.fori_loop(0, CPS, _chunk, 0)
        return carry

    lax.fori_loop(0, NSEG, _seg, 0)
    plsc.subcore_barrier()
    for ss in range(NS):
        @pl.when(s == ss)
        def _():
            pltpu.sync_copy(acc_s.at[pl.ds(ss * RPT, RPT)],
                            acc_hbm.at[c, pl.ds(ss * RPT, RPT)])


def _edge_pass(haug, a_s, a_d, src_pad, dst_pad):
    w_pad = pl.kernel(
        _sc_w_body,
        out_type=jax.ShapeDtypeStruct((E_PAD,), jnp.float32),
        mesh=_sc_mesh(),
        compiler_params=_SC_PARAMS,
        scratch_types=[
            pltpu.VMEM((T,), jnp.int32),
            pltpu.VMEM((T,), jnp.int32),
            pltpu.VMEM((N,), jnp.float32),
            pltpu.VMEM((N,), jnp.float32),
            pltpu.VMEM((T,), jnp.float32),
        ],
    )(a_s, a_d, src_pad, dst_pad)
    acc = pl.kernel(
        _sc_msg_body,
        out_type=jax.ShapeDtypeStruct((NC, NPAD, FW), jnp.float32),
        mesh=_sc_mesh(),
        compiler_params=_SC_PARAMS,
        scratch_types=[
            pltpu.VMEM((S,), jnp.int32),
            pltpu.VMEM((S,), jnp.int32),
            pltpu.VMEM((S,), jnp.float32),
            pltpu.VMEM((C,), jnp.int32),
            pltpu.VMEM((C,), jnp.int32),
            pltpu.VMEM((C,), jnp.float32),
            pltpu.VMEM((C, FW), jnp.float32),
            pltpu.VMEM_SHARED((NPAD, FW), jnp.float32),
            pltpu.SemaphoreType.DMA,
        ],
    )(haug, w_pad, src_pad, dst_pad)
    return acc


def kernel(x, edge_index, batch,
           W0, asrc0, adst0, b0,
           W1, asrc1, adst1, b1,
           W2, asrc2, adst2, b2,
           W_lin, b_lin):
    pad = jnp.zeros((E_PAD - E,), jnp.int32)
    src_pad = jnp.concatenate([edge_index[0], pad])
    dst_pad = jnp.concatenate([edge_index[1], pad])

    h, a_s, a_d, ws = _layer_in(x, W0, asrc0, adst0)
    acc = _edge_pass(h, a_s.reshape(N), a_d.reshape(N), src_pad, dst_pad)
    h, a_s, a_d, ws = _combine(acc, ws, h, b0.reshape(1, F), W1, asrc1, adst1)
    acc = _edge_pass(h, a_s.reshape(N), a_d.reshape(N), src_pad, dst_pad)
    h, a_s, a_d, ws = _combine(acc, ws, h, b1.reshape(1, F), W2, asrc2, adst2)
    acc = _edge_pass(h, a_s.reshape(N), a_d.reshape(N), src_pad, dst_pad)
    _, _, res = _final(acc, ws, h, b2.reshape(1, F), W_lin,
                       b_lin.reshape(1, 1), batch.reshape(N, 1))
    return res.reshape(G)


# speedup vs baseline: 13.9006x; 1.0785x over previous
import jax
import jax.numpy as jnp
from jax import lax
from jax.experimental import pallas as pl
from jax.experimental.pallas import tpu as pltpu
from jax.experimental.pallas import tpu_sc as plsc

N = 10000
E = 320000
F = 128
G = 64

NC = 2
NS = 16
NW = NC * NS
T = 10240
E_PAD = T * NW
C = 128
S = 2048
NSEG = T // S
CB = 64
PAIRS = S // (2 * CB)
NCH = T // C
FW = F + 16
NPAD = 10240
RPT = NPAD // NS

BN = 1000
NB = N // BN

_SC_PARAMS = pltpu.CompilerParams(needs_layout_passes=False,
                                  use_tc_tiling_on_sc=False)


def _aug(h):
    n = h.shape[0]
    return jnp.concatenate(
        [h, jnp.ones((n, 1), jnp.float32), jnp.zeros((n, FW - F - 1),
                                                     jnp.float32)], axis=1)


def _attn_tail(h, as_w, ad_w, as_ref, ad_ref, ws_ref):
    a_s = jnp.sum(h * as_w, axis=1, keepdims=True)
    a_d = jnp.sum(h * ad_w, axis=1, keepdims=True)
    as_ref[...] = a_s
    ad_ref[...] = a_d
    al = a_s + a_d
    al = jnp.maximum(al, 0.2 * al)
    ws_ref[...] = jnp.exp(al)


def _mm_attn_body(x_ref, W_ref, as_w_ref, ad_w_ref,
                  h_ref, as_ref, ad_ref, ws_ref):
    h = jnp.dot(x_ref[...], W_ref[...], preferred_element_type=jnp.float32,
                 precision=lax.Precision.HIGHEST)
    h_ref[...] = _aug(h)
    _attn_tail(h, as_w_ref[...], ad_w_ref[...], as_ref, ad_ref, ws_ref)


def _layer_in(x, W, asrc, adst):
    return pl.pallas_call(
        _mm_attn_body,
        grid=(NB,),
        in_specs=[
            pl.BlockSpec((BN, F), lambda i: (i, 0)),
            pl.BlockSpec((F, F), lambda i: (0, 0)),
            pl.BlockSpec((1, F), lambda i: (0, 0)),
            pl.BlockSpec((1, F), lambda i: (0, 0)),
        ],
        out_specs=[
            pl.BlockSpec((BN, FW), lambda i: (i, 0)),
            pl.BlockSpec((BN, 1), lambda i: (i, 0)),
            pl.BlockSpec((BN, 1), lambda i: (i, 0)),
            pl.BlockSpec((BN, 1), lambda i: (i, 0)),
        ],
        out_shape=[
            jax.ShapeDtypeStruct((N, FW), jnp.float32),
            jax.ShapeDtypeStruct((N, 1), jnp.float32),
            jax.ShapeDtypeStruct((N, 1), jnp.float32),
            jax.ShapeDtypeStruct((N, 1), jnp.float32),
        ],
    )(x, W, asrc, adst)


def _node_out(acc, ws, haug, b):
    h = haug[:, :F]
    num = acc[0, :, :F] + acc[1, :, :F] + ws * h
    d = acc[0, :, F:F + 1] + acc[1, :, F:F + 1] + ws
    xn = num / d + b
    return jnp.where(xn > 0, xn, jnp.exp(jnp.minimum(xn, 0.0)) - 1.0)


def _combine_body(acc_ref, ws_ref, h_ref, b_ref, W_ref,
                  as_w_ref, ad_w_ref,
                  h2_ref, as_ref, ad_ref, ws2_ref):
    xn = _node_out(acc_ref[...], ws_ref[...], h_ref[...], b_ref[...])
    h2 = jnp.dot(xn, W_ref[...], preferred_element_type=jnp.float32,
                 precision=lax.Precision.HIGHEST)
    h2_ref[...] = _aug(h2)
    _attn_tail(h2, as_w_ref[...], ad_w_ref[...], as_ref, ad_ref, ws2_ref)


def _combine(acc, ws, haug, b, W, asrc, adst):
    return pl.pallas_call(
        _combine_body,
        grid=(NB,),
        in_specs=[
            pl.BlockSpec((2, BN, FW), lambda i: (0, i, 0)),
            pl.BlockSpec((BN, 1), lambda i: (i, 0)),
            pl.BlockSpec((BN, FW), lambda i: (i, 0)),
            pl.BlockSpec((1, F), lambda i: (0, 0)),
            pl.BlockSpec((F, F), lambda i: (0, 0)),
            pl.BlockSpec((1, F), lambda i: (0, 0)),
            pl.BlockSpec((1, F), lambda i: (0, 0)),
        ],
        out_specs=[
            pl.BlockSpec((BN, FW), lambda i: (i, 0)),
            pl.BlockSpec((BN, 1), lambda i: (i, 0)),
            pl.BlockSpec((BN, 1), lambda i: (i, 0)),
            pl.BlockSpec((BN, 1), lambda i: (i, 0)),
        ],
        out_shape=[
            jax.ShapeDtypeStruct((N, FW), jnp.float32),
            jax.ShapeDtypeStruct((N, 1), jnp.float32),
            jax.ShapeDtypeStruct((N, 1), jnp.float32),
            jax.ShapeDtypeStruct((N, 1), jnp.float32),
        ],
    )(acc, ws, haug, b, W, asrc, adst)


def _final_body(acc_ref, ws_ref, h_ref, b_ref, Wl_ref, bl_ref,
                batch_ref, sums_ref, cnts_ref, res_ref):
    i = pl.program_id(0)
    xn = _node_out(acc_ref[...], ws_ref[...], h_ref[...], b_ref[...])
    z = jnp.dot(xn, Wl_ref[...], preferred_element_type=jnp.float32,
                 precision=lax.Precision.HIGHEST)
    gid = lax.broadcasted_iota(jnp.int32, (1, G), 1)
    m = (batch_ref[...] == gid).astype(jnp.float32)
    part = jnp.sum(m * z, axis=0, keepdims=True)
    cnt = jnp.sum(m, axis=0, keepdims=True)

    @pl.when(i == 0)
    def _():
        sums_ref[...] = part
        cnts_ref[...] = cnt

    @pl.when(i > 0)
    def _():
        sums_ref[...] = sums_ref[...] + part
        cnts_ref[...] = cnts_ref[...] + cnt

    @pl.when(i == NB - 1)
    def _():
        res_ref[...] = (sums_ref[...] / jnp.maximum(cnts_ref[...], 1.0)
                        + bl_ref[...])


def _final(acc, ws, haug, b, W_lin, b_lin, batch2d):
    return pl.pallas_call(
        _final_body,
        grid=(NB,),
        in_specs=[
            pl.BlockSpec((2, BN, FW), lambda i: (0, i, 0)),
            pl.BlockSpec((BN, 1), lambda i: (i, 0)),
            pl.BlockSpec((BN, FW), lambda i: (i, 0)),
            pl.BlockSpec((1, F), lambda i: (0, 0)),
            pl.BlockSpec((F, 1), lambda i: (0, 0)),
            pl.BlockSpec((1, 1), lambda i: (0, 0)),
            pl.BlockSpec((BN, 1), lambda i: (i, 0)),
        ],
        out_specs=[
            pl.BlockSpec((1, G), lambda i: (0, 0)),
            pl.BlockSpec((1, G), lambda i: (0, 0)),
            pl.BlockSpec((1, G), lambda i: (0, 0)),
        ],
        out_shape=[
            jax.ShapeDtypeStruct((1, G), jnp.float32),
            jax.ShapeDtypeStruct((1, G), jnp.float32),
            jax.ShapeDtypeStruct((1, G), jnp.float32),
        ],
    )(acc, ws, haug, b, W_lin, b_lin, batch2d)


def _sc_mesh():
    return plsc.VectorSubcoreMesh(core_axis_name="c", subcore_axis_name="s",
                                  num_cores=NC, num_subcores=NS)


def _sc_w_body(as_hbm, ad_hbm, src_hbm, dst_hbm,
               w_hbm,
               src_t, dst_t, asv, adv, w_t):
    c = lax.axis_index("c")
    s = lax.axis_index("s")
    wid = s * NC + c
    lane = lax.iota(jnp.int32, 16)

    pltpu.sync_copy(src_hbm.at[pl.ds(wid * T, T)], src_t)
    pltpu.sync_copy(dst_hbm.at[pl.ds(wid * T, T)], dst_t)
    pltpu.sync_copy(as_hbm, asv)
    pltpu.sync_copy(ad_hbm, adv)

    def _chunk(k, carry):
        base = k * C
        for v in range(C // 16):
            s16 = src_t[pl.ds(base + v * 16, 16)]
            d16 = dst_t[pl.ds(base + v * 16, 16)]
            al = (plsc.load_gather(asv, [s16])
                  + plsc.load_gather(adv, [d16]))
            al = jnp.maximum(al, 0.2 * al)
            w = jnp.exp(al)
            eid = wid * T + base + v * 16 + lane
            w = jnp.where(eid < E, w, 0.0)
            w_t[pl.ds(base + v * 16, 16)] = w
        return carry

    lax.fori_loop(0, NCH, _chunk, 0)
    pltpu.sync_copy(w_t, w_hbm.at[pl.ds(wid * T, T)])


def _sc_msg_body(h_hbm, w_hbm, src_hbm, dst_hbm,
                 acc_hbm,
                 src_g, dst_g, w_g,
                 srcbuf0, dstbuf0, wbuf0, rows0,
                 srcbuf1, dstbuf1, wbuf1, rows1,
                 acc_s, sg0, sg1, ss0, ss1):
    c = lax.axis_index("c")
    s = lax.axis_index("s")
    wid = s * NC + c
    zero16 = jnp.zeros((16,), jnp.float32)

    def _zr(rows):
        def _z(i, carry):
            for q in range(FW // 16):
                rows[i, pl.ds(q * 16, 16)] = zero16
            return carry
        lax.fori_loop(0, CB, _z, 0)

    _zr(rows0)
    _zr(rows1)
    for ss_ in range(NS):
        @pl.when(s == ss_)
        def _():
            for j in range(RPT // (2 * CB)):
                pltpu.sync_copy(rows0,
                                acc_s.at[pl.ds(ss_ * RPT + 2 * j * CB, CB)])
                pltpu.sync_copy(rows1,
                                acc_s.at[pl.ds(ss_ * RPT + (2 * j + 1) * CB,
                                               CB)])
    plsc.subcore_barrier()

    def _fill(base, srcbuf, dstbuf, wbuf):
        for v in range(CB // 16):
            srcbuf[pl.ds(v * 16, 16)] = src_g[pl.ds(base + v * 16, 16)]
            dstbuf[pl.ds(v * 16, 16)] = dst_g[pl.ds(base + v * 16, 16)]
            wbuf[pl.ds(v * 16, 16)] = w_g[pl.ds(base + v * 16, 16)]

    def _scale(rows, wbuf):
        def _sc(j, carry):
            wj = plsc.load_gather(
                wbuf, [jnp.broadcast_to(j, (16,)).astype(jnp.int32)])
            for q in range(FW // 16):
                rows[j, pl.ds(q * 16, 16)] = rows[j, pl.ds(q * 16, 16)] * wj
            return carry
        lax.fori_loop(0, CB, _sc, 0)

    def _seg(g, carry):
        off = wid * T + g * S
        pltpu.sync_copy(src_hbm.at[pl.ds(off, S)], src_g)
        pltpu.sync_copy(dst_hbm.at[pl.ds(off, S)], dst_g)
        pltpu.sync_copy(w_hbm.at[pl.ds(off, S)], w_g)

        def _pair(kk, carry2):
            base0 = kk * 2 * CB
            base1 = base0 + CB
            _fill(base0, srcbuf0, dstbuf0, wbuf0)
            g0 = pltpu.async_copy(h_hbm.at[srcbuf0], rows0, sg0)
            _fill(base1, srcbuf1, dstbuf1, wbuf1)
            g1 = pltpu.async_copy(h_hbm.at[srcbuf1], rows1, sg1)
            g0.wait()
            _scale(rows0, wbuf0)
            sc0 = pltpu.async_copy(rows0, acc_s.at[dstbuf0], ss0, add=True)
            g1.wait()
            _scale(rows1, wbuf1)
            sc1 = pltpu.async_copy(rows1, acc_s.at[dstbuf1], ss1, add=True)
            sc0.wait()
            sc1.wait()
            return carry2

        lax.fori_loop(0, PAIRS, _pair, 0)
        return carry

    lax.fori_loop(0, NSEG, _seg, 0)
    plsc.subcore_barrier()
    for ss_ in range(NS):
        @pl.when(s == ss_)
        def _():
            pltpu.sync_copy(acc_s.at[pl.ds(ss_ * RPT, RPT)],
                            acc_hbm.at[c, pl.ds(ss_ * RPT, RPT)])


def _edge_pass(haug, a_s, a_d, src_pad, dst_pad):
    w_pad = pl.kernel(
        _sc_w_body,
        out_type=jax.ShapeDtypeStruct((E_PAD,), jnp.float32),
        mesh=_sc_mesh(),
        compiler_params=_SC_PARAMS,
        scratch_types=[
            pltpu.VMEM((T,), jnp.int32),
            pltpu.VMEM((T,), jnp.int32),
            pltpu.VMEM((N,), jnp.float32),
            pltpu.VMEM((N,), jnp.float32),
            pltpu.VMEM((T,), jnp.float32),
        ],
    )(a_s, a_d, src_pad, dst_pad)
    acc = pl.kernel(
        _sc_msg_body,
        out_type=jax.ShapeDtypeStruct((NC, NPAD, FW), jnp.float32),
        mesh=_sc_mesh(),
        compiler_params=_SC_PARAMS,
        scratch_types=[
            pltpu.VMEM((S,), jnp.int32),
            pltpu.VMEM((S,), jnp.int32),
            pltpu.VMEM((S,), jnp.float32),
            pltpu.VMEM((CB,), jnp.int32),
            pltpu.VMEM((CB,), jnp.int32),
            pltpu.VMEM((CB,), jnp.float32),
            pltpu.VMEM((CB, FW), jnp.float32),
            pltpu.VMEM((CB,), jnp.int32),
            pltpu.VMEM((CB,), jnp.int32),
            pltpu.VMEM((CB,), jnp.float32),
            pltpu.VMEM((CB, FW), jnp.float32),
            pltpu.VMEM_SHARED((NPAD, FW), jnp.float32),
            pltpu.SemaphoreType.DMA,
            pltpu.SemaphoreType.DMA,
            pltpu.SemaphoreType.DMA,
            pltpu.SemaphoreType.DMA,
        ],
    )(haug, w_pad, src_pad, dst_pad)
    return acc


def kernel(x, edge_index, batch,
           W0, asrc0, adst0, b0,
           W1, asrc1, adst1, b1,
           W2, asrc2, adst2, b2,
           W_lin, b_lin):
    pad = jnp.zeros((E_PAD - E,), jnp.int32)
    src_pad = jnp.concatenate([edge_index[0], pad])
    dst_pad = jnp.concatenate([edge_index[1], pad])

    h, a_s, a_d, ws = _layer_in(x, W0, asrc0, adst0)
    acc = _edge_pass(h, a_s.reshape(N), a_d.reshape(N), src_pad, dst_pad)
    h, a_s, a_d, ws = _combine(acc, ws, h, b0.reshape(1, F), W1, asrc1, adst1)
    acc = _edge_pass(h, a_s.reshape(N), a_d.reshape(N), src_pad, dst_pad)
    h, a_s, a_d, ws = _combine(acc, ws, h, b1.reshape(1, F), W2, asrc2, adst2)
    acc = _edge_pass(h, a_s.reshape(N), a_d.reshape(N), src_pad, dst_pad)
    _, _, res = _final(acc, ws, h, b2.reshape(1, F), W_lin,
                       b_lin.reshape(1, 1), batch.reshape(N, 1))
    return res.reshape(G)


# speedup vs baseline: 14.8169x; 1.0659x over previous
import jax
import jax.numpy as jnp
from jax import lax
from jax.experimental import pallas as pl
from jax.experimental.pallas import tpu as pltpu
from jax.experimental.pallas import tpu_sc as plsc

N = 10000
E = 320000
F = 128
G = 64

NC = 2
NS = 16
NW = NC * NS
T = 10240
E_PAD = T * NW
C = 128
S = 2048
NSEG = T // S
CB = 64
PAIRS = S // (2 * CB)
NCH = T // C
FW = F + 16
NPAD = 10240
RPT = NPAD // NS

BN = 1000
NB = N // BN

_SC_PARAMS = pltpu.CompilerParams(needs_layout_passes=False,
                                  use_tc_tiling_on_sc=False)


def _aug(h):
    n = h.shape[0]
    return jnp.concatenate(
        [h, jnp.ones((n, 1), jnp.float32), jnp.zeros((n, FW - F - 1),
                                                     jnp.float32)], axis=1)


def _attn_tail(h, as_w, ad_w, as_ref, ad_ref, ws_ref):
    a_s = jnp.sum(h * as_w, axis=1, keepdims=True)
    a_d = jnp.sum(h * ad_w, axis=1, keepdims=True)
    as_ref[...] = a_s
    ad_ref[...] = a_d
    al = a_s + a_d
    al = jnp.maximum(al, 0.2 * al)
    ws_ref[...] = jnp.exp(al)


def _mm_attn_body(x_ref, W_ref, as_w_ref, ad_w_ref,
                  h_ref, as_ref, ad_ref, ws_ref):
    h = jnp.dot(x_ref[...], W_ref[...], preferred_element_type=jnp.float32,
                 precision=lax.Precision.HIGHEST)
    h_ref[...] = _aug(h)
    _attn_tail(h, as_w_ref[...], ad_w_ref[...], as_ref, ad_ref, ws_ref)


def _layer_in(x, W, asrc, adst):
    return pl.pallas_call(
        _mm_attn_body,
        grid=(NB,),
        in_specs=[
            pl.BlockSpec((BN, F), lambda i: (i, 0)),
            pl.BlockSpec((F, F), lambda i: (0, 0)),
            pl.BlockSpec((1, F), lambda i: (0, 0)),
            pl.BlockSpec((1, F), lambda i: (0, 0)),
        ],
        out_specs=[
            pl.BlockSpec((BN, FW), lambda i: (i, 0)),
            pl.BlockSpec((BN, 1), lambda i: (i, 0)),
            pl.BlockSpec((BN, 1), lambda i: (i, 0)),
            pl.BlockSpec((BN, 1), lambda i: (i, 0)),
        ],
        out_shape=[
            jax.ShapeDtypeStruct((N, FW), jnp.float32),
            jax.ShapeDtypeStruct((N, 1), jnp.float32),
            jax.ShapeDtypeStruct((N, 1), jnp.float32),
            jax.ShapeDtypeStruct((N, 1), jnp.float32),
        ],
    )(x, W, asrc, adst)


def _node_out(acc, ws, haug, b):
    h = haug[:, :F]
    num = acc[0, :, :F] + acc[1, :, :F] + ws * h
    d = acc[0, :, F:F + 1] + acc[1, :, F:F + 1] + ws
    xn = num / d + b
    return jnp.where(xn > 0, xn, jnp.exp(jnp.minimum(xn, 0.0)) - 1.0)


def _combine_body(acc_ref, ws_ref, h_ref, b_ref, W_ref,
                  as_w_ref, ad_w_ref,
                  h2_ref, as_ref, ad_ref, ws2_ref):
    xn = _node_out(acc_ref[...], ws_ref[...], h_ref[...], b_ref[...])
    h2 = jnp.dot(xn, W_ref[...], preferred_element_type=jnp.float32,
                 precision=lax.Precision.HIGHEST)
    h2_ref[...] = _aug(h2)
    _attn_tail(h2, as_w_ref[...], ad_w_ref[...], as_ref, ad_ref, ws2_ref)


def _combine(acc, ws, haug, b, W, asrc, adst):
    return pl.pallas_call(
        _combine_body,
        grid=(NB,),
        in_specs=[
            pl.BlockSpec((2, BN, FW), lambda i: (0, i, 0)),
            pl.BlockSpec((BN, 1), lambda i: (i, 0)),
            pl.BlockSpec((BN, FW), lambda i: (i, 0)),
            pl.BlockSpec((1, F), lambda i: (0, 0)),
            pl.BlockSpec((F, F), lambda i: (0, 0)),
            pl.BlockSpec((1, F), lambda i: (0, 0)),
            pl.BlockSpec((1, F), lambda i: (0, 0)),
        ],
        out_specs=[
            pl.BlockSpec((BN, FW), lambda i: (i, 0)),
            pl.BlockSpec((BN, 1), lambda i: (i, 0)),
            pl.BlockSpec((BN, 1), lambda i: (i, 0)),
            pl.BlockSpec((BN, 1), lambda i: (i, 0)),
        ],
        out_shape=[
            jax.ShapeDtypeStruct((N, FW), jnp.float32),
            jax.ShapeDtypeStruct((N, 1), jnp.float32),
            jax.ShapeDtypeStruct((N, 1), jnp.float32),
            jax.ShapeDtypeStruct((N, 1), jnp.float32),
        ],
    )(acc, ws, haug, b, W, asrc, adst)


def _final_body(acc_ref, ws_ref, h_ref, b_ref, Wl_ref, bl_ref,
                batch_ref, sums_ref, cnts_ref, res_ref):
    i = pl.program_id(0)
    xn = _node_out(acc_ref[...], ws_ref[...], h_ref[...], b_ref[...])
    z = jnp.dot(xn, Wl_ref[...], preferred_element_type=jnp.float32,
                 precision=lax.Precision.HIGHEST)
    gid = lax.broadcasted_iota(jnp.int32, (1, G), 1)
    m = (batch_ref[...] == gid).astype(jnp.float32)
    part = jnp.sum(m * z, axis=0, keepdims=True)
    cnt = jnp.sum(m, axis=0, keepdims=True)

    @pl.when(i == 0)
    def _():
        sums_ref[...] = part
        cnts_ref[...] = cnt

    @pl.when(i > 0)
    def _():
        sums_ref[...] = sums_ref[...] + part
        cnts_ref[...] = cnts_ref[...] + cnt

    @pl.when(i == NB - 1)
    def _():
        res_ref[...] = (sums_ref[...] / jnp.maximum(cnts_ref[...], 1.0)
                        + bl_ref[...])


def _final(acc, ws, haug, b, W_lin, b_lin, batch2d):
    return pl.pallas_call(
        _final_body,
        grid=(NB,),
        in_specs=[
            pl.BlockSpec((2, BN, FW), lambda i: (0, i, 0)),
            pl.BlockSpec((BN, 1), lambda i: (i, 0)),
            pl.BlockSpec((BN, FW), lambda i: (i, 0)),
            pl.BlockSpec((1, F), lambda i: (0, 0)),
            pl.BlockSpec((F, 1), lambda i: (0, 0)),
            pl.BlockSpec((1, 1), lambda i: (0, 0)),
            pl.BlockSpec((BN, 1), lambda i: (i, 0)),
        ],
        out_specs=[
            pl.BlockSpec((1, G), lambda i: (0, 0)),
            pl.BlockSpec((1, G), lambda i: (0, 0)),
            pl.BlockSpec((1, G), lambda i: (0, 0)),
        ],
        out_shape=[
            jax.ShapeDtypeStruct((1, G), jnp.float32),
            jax.ShapeDtypeStruct((1, G), jnp.float32),
            jax.ShapeDtypeStruct((1, G), jnp.float32),
        ],
    )(acc, ws, haug, b, W_lin, b_lin, batch2d)


def _sc_mesh():
    return plsc.VectorSubcoreMesh(core_axis_name="c", subcore_axis_name="s",
                                  num_cores=NC, num_subcores=NS)


def _sc_w_body(as_hbm, ad_hbm, src_hbm, dst_hbm,
               w_hbm,
               src_t, dst_t, asv, adv, w_t):
    c = lax.axis_index("c")
    s = lax.axis_index("s")
    wid = s * NC + c
    lane = lax.iota(jnp.int32, 16)

    pltpu.sync_copy(src_hbm.at[pl.ds(wid * T, T)], src_t)
    pltpu.sync_copy(dst_hbm.at[pl.ds(wid * T, T)], dst_t)
    pltpu.sync_copy(as_hbm, asv)
    pltpu.sync_copy(ad_hbm, adv.at[pl.ds(0, N)])

    def _chunk(k, carry):
        base = k * C
        for v in range(C // 16):
            s16 = src_t[pl.ds(base + v * 16, 16)]
            d16 = dst_t[pl.ds(base + v * 16, 16)]
            al = (plsc.load_gather(asv, [s16])
                  + plsc.load_gather(adv, [d16]))
            al = jnp.maximum(al, 0.2 * al)
            w = jnp.exp(al)
            eid = wid * T + base + v * 16 + lane
            w = jnp.where(eid < E, w, 0.0)
            w_t[pl.ds(base + v * 16, 16)] = w
        return carry

    lax.fori_loop(0, NCH, _chunk, 0)
    pltpu.sync_copy(w_t, w_hbm.at[pl.ds(wid * T, T)])


def _sc_msg_body(h_hbm, w_hbm, src_hbm, dst_hbm,
                 acc_hbm,
                 src_g, dst_g, w_g,
                 srcbuf0, dstbuf0, wbuf0, rows0,
                 srcbuf1, dstbuf1, wbuf1, rows1,
                 acc_s, sg0, sg1, ss0, ss1):
    c = lax.axis_index("c")
    s = lax.axis_index("s")
    wid = s * NC + c
    zero16 = jnp.zeros((16,), jnp.float32)

    def _zr(rows):
        def _z(i, carry):
            for q in range(FW // 16):
                rows[i, pl.ds(q * 16, 16)] = zero16
            return carry
        lax.fori_loop(0, CB, _z, 0)

    _zr(rows0)
    _zr(rows1)
    for ss_ in range(NS):
        @pl.when(s == ss_)
        def _():
            for j in range(RPT // (2 * CB)):
                pltpu.sync_copy(rows0,
                                acc_s.at[pl.ds(ss_ * RPT + 2 * j * CB, CB)])
                pltpu.sync_copy(rows1,
                                acc_s.at[pl.ds(ss_ * RPT + (2 * j + 1) * CB,
                                               CB)])
    plsc.subcore_barrier()

    def _fill(base, srcbuf, dstbuf, wbuf):
        for v in range(CB // 16):
            srcbuf[pl.ds(v * 16, 16)] = src_g[pl.ds(base + v * 16, 16)]
            dstbuf[pl.ds(v * 16, 16)] = dst_g[pl.ds(base + v * 16, 16)]
            wbuf[pl.ds(v * 16, 16)] = w_g[pl.ds(base + v * 16, 16)]

    def _scale(rows, wbuf):
        def _sc(j, carry):
            wj = plsc.load_gather(
                wbuf, [jnp.broadcast_to(j, (16,)).astype(jnp.int32)])
            for q in range(FW // 16):
                rows[j, pl.ds(q * 16, 16)] = rows[j, pl.ds(q * 16, 16)] * wj
            return carry
        lax.fori_loop(0, CB, _sc, 0)

    def _seg(g, carry):
        off = wid * T + g * S
        pltpu.sync_copy(src_hbm.at[pl.ds(off, S)], src_g)
        pltpu.sync_copy(dst_hbm.at[pl.ds(off, S)], dst_g)
        pltpu.sync_copy(w_hbm.at[pl.ds(off, S)], w_g)

        def _pair(kk, carry2):
            base0 = kk * 2 * CB
            base1 = base0 + CB
            _fill(base0, srcbuf0, dstbuf0, wbuf0)
            g0 = pltpu.async_copy(h_hbm.at[srcbuf0], rows0, sg0)
            _fill(base1, srcbuf1, dstbuf1, wbuf1)
            g1 = pltpu.async_copy(h_hbm.at[srcbuf1], rows1, sg1)
            g0.wait()
            _scale(rows0, wbuf0)
            sc0 = pltpu.async_copy(rows0, acc_s.at[dstbuf0], ss0, add=True)
            g1.wait()
            _scale(rows1, wbuf1)
            sc1 = pltpu.async_copy(rows1, acc_s.at[dstbuf1], ss1, add=True)
            sc0.wait()
            sc1.wait()
            return carry2

        lax.fori_loop(0, PAIRS, _pair, 0)
        return carry

    lax.fori_loop(0, NSEG, _seg, 0)
    plsc.subcore_barrier()
    for ss_ in range(NS):
        @pl.when(s == ss_)
        def _():
            pltpu.sync_copy(acc_s.at[pl.ds(ss_ * RPT, RPT)],
                            acc_hbm.at[c, pl.ds(ss_ * RPT, RPT)])


def _edge_pass(haug, a_s, a_d, src_pad, dst_pad):
    w_pad = pl.kernel(
        _sc_w_body,
        out_type=jax.ShapeDtypeStruct((E_PAD,), jnp.float32),
        mesh=_sc_mesh(),
        compiler_params=_SC_PARAMS,
        scratch_types=[
            pltpu.VMEM((T,), jnp.int32),
            pltpu.VMEM((T,), jnp.int32),
            pltpu.VMEM((N,), jnp.float32),
            pltpu.VMEM((NPAD,), jnp.float32),
            pltpu.VMEM((T,), jnp.float32),
        ],
    )(a_s, a_d, src_pad, dst_pad)
    acc = pl.kernel(
        _sc_msg_body,
        out_type=jax.ShapeDtypeStruct((NC, NPAD, FW), jnp.float32),
        mesh=_sc_mesh(),
        compiler_params=_SC_PARAMS,
        scratch_types=[
            pltpu.VMEM((S,), jnp.int32),
            pltpu.VMEM((S,), jnp.int32),
            pltpu.VMEM((S,), jnp.float32),
            pltpu.VMEM((CB,), jnp.int32),
            pltpu.VMEM((CB,), jnp.int32),
            pltpu.VMEM((CB,), jnp.float32),
            pltpu.VMEM((CB, FW), jnp.float32),
            pltpu.VMEM((CB,), jnp.int32),
            pltpu.VMEM((CB,), jnp.int32),
            pltpu.VMEM((CB,), jnp.float32),
            pltpu.VMEM((CB, FW), jnp.float32),
            pltpu.VMEM_SHARED((NPAD, FW), jnp.float32),
            pltpu.SemaphoreType.DMA,
            pltpu.SemaphoreType.DMA,
            pltpu.SemaphoreType.DMA,
            pltpu.SemaphoreType.DMA,
        ],
    )(haug, w_pad, src_pad, dst_pad)
    return acc


def kernel(x, edge_index, batch,
           W0, asrc0, adst0, b0,
           W1, asrc1, adst1, b1,
           W2, asrc2, adst2, b2,
           W_lin, b_lin):
    pad = jnp.zeros((E_PAD - E,), jnp.int32)
    src_pad = jnp.concatenate([edge_index[0], pad])
    padd = N + (jnp.arange(E_PAD - E, dtype=jnp.int32) % (NPAD - N))
    dst_pad = jnp.concatenate([edge_index[1], padd])

    h, a_s, a_d, ws = _layer_in(x, W0, asrc0, adst0)
    acc = _edge_pass(h, a_s.reshape(N), a_d.reshape(N), src_pad, dst_pad)
    h, a_s, a_d, ws = _combine(acc, ws, h, b0.reshape(1, F), W1, asrc1, adst1)
    acc = _edge_pass(h, a_s.reshape(N), a_d.reshape(N), src_pad, dst_pad)
    h, a_s, a_d, ws = _combine(acc, ws, h, b1.reshape(1, F), W2, asrc2, adst2)
    acc = _edge_pass(h, a_s.reshape(N), a_d.reshape(N), src_pad, dst_pad)
    _, _, res = _final(acc, ws, h, b2.reshape(1, F), W_lin,
                       b_lin.reshape(1, 1), batch.reshape(N, 1))
    return res.reshape(G)


# speedup vs baseline: 14.8688x; 1.0035x over previous
import jax
import jax.numpy as jnp
from jax import lax
from jax.experimental import pallas as pl
from jax.experimental.pallas import tpu as pltpu
from jax.experimental.pallas import tpu_sc as plsc

N = 10000
E = 320000
F = 128
G = 64

NC = 2
NS = 16
NW = NC * NS
T = 10240
E_PAD = T * NW
C = 128
S = 2048
NSEG = T // S
CB = 64
PAIRS = S // (2 * CB)
NCH = T // C
FW = F + 16
NPAD = 10240
RPT = NPAD // NS

BN = 1000
NB = N // BN

_SC_PARAMS = pltpu.CompilerParams(needs_layout_passes=False,
                                  use_tc_tiling_on_sc=False)


def _aug(h):
    n = h.shape[0]
    return jnp.concatenate(
        [h, jnp.ones((n, 1), jnp.float32), jnp.zeros((n, FW - F - 1),
                                                     jnp.float32)], axis=1)


def _attn_tail(h, as_w, ad_w, as_ref, ad_ref, ws_ref):
    a_s = jnp.sum(h * as_w, axis=1, keepdims=True)
    a_d = jnp.sum(h * ad_w, axis=1, keepdims=True)
    as_ref[...] = a_s
    ad_ref[...] = a_d
    al = a_s + a_d
    al = jnp.maximum(al, 0.2 * al)
    ws_ref[...] = jnp.exp(al)


def _mm_attn_body(x_ref, W_ref, as_w_ref, ad_w_ref,
                  h_ref, as_ref, ad_ref, ws_ref):
    h = jnp.dot(x_ref[...], W_ref[...], preferred_element_type=jnp.float32,
                 precision=lax.Precision.HIGHEST)
    h_ref[...] = _aug(h)
    _attn_tail(h, as_w_ref[...], ad_w_ref[...], as_ref, ad_ref, ws_ref)


def _layer_in(x, W, asrc, adst):
    return pl.pallas_call(
        _mm_attn_body,
        grid=(NB,),
        in_specs=[
            pl.BlockSpec((BN, F), lambda i: (i, 0)),
            pl.BlockSpec((F, F), lambda i: (0, 0)),
            pl.BlockSpec((1, F), lambda i: (0, 0)),
            pl.BlockSpec((1, F), lambda i: (0, 0)),
        ],
        out_specs=[
            pl.BlockSpec((BN, FW), lambda i: (i, 0)),
            pl.BlockSpec((BN, 1), lambda i: (i, 0)),
            pl.BlockSpec((BN, 1), lambda i: (i, 0)),
            pl.BlockSpec((BN, 1), lambda i: (i, 0)),
        ],
        out_shape=[
            jax.ShapeDtypeStruct((N, FW), jnp.float32),
            jax.ShapeDtypeStruct((N, 1), jnp.float32),
            jax.ShapeDtypeStruct((N, 1), jnp.float32),
            jax.ShapeDtypeStruct((N, 1), jnp.float32),
        ],
    )(x, W, asrc, adst)


def _node_out(acc, ws, haug, b):
    h = haug[:, :F]
    num = acc[0, :, :F] + acc[1, :, :F] + ws * h
    d = acc[0, :, F:F + 1] + acc[1, :, F:F + 1] + ws
    xn = num / d + b
    return jnp.where(xn > 0, xn, jnp.exp(jnp.minimum(xn, 0.0)) - 1.0)


def _combine_body(acc_ref, ws_ref, h_ref, b_ref, W_ref,
                  as_w_ref, ad_w_ref,
                  h2_ref, as_ref, ad_ref, ws2_ref):
    xn = _node_out(acc_ref[...], ws_ref[...], h_ref[...], b_ref[...])
    h2 = jnp.dot(xn, W_ref[...], preferred_element_type=jnp.float32,
                 precision=lax.Precision.HIGHEST)
    h2_ref[...] = _aug(h2)
    _attn_tail(h2, as_w_ref[...], ad_w_ref[...], as_ref, ad_ref, ws2_ref)


def _combine(acc, ws, haug, b, W, asrc, adst):
    return pl.pallas_call(
        _combine_body,
        grid=(NB,),
        in_specs=[
            pl.BlockSpec((2, BN, FW), lambda i: (0, i, 0)),
            pl.BlockSpec((BN, 1), lambda i: (i, 0)),
            pl.BlockSpec((BN, FW), lambda i: (i, 0)),
            pl.BlockSpec((1, F), lambda i: (0, 0)),
            pl.BlockSpec((F, F), lambda i: (0, 0)),
            pl.BlockSpec((1, F), lambda i: (0, 0)),
            pl.BlockSpec((1, F), lambda i: (0, 0)),
        ],
        out_specs=[
            pl.BlockSpec((BN, FW), lambda i: (i, 0)),
            pl.BlockSpec((BN, 1), lambda i: (i, 0)),
            pl.BlockSpec((BN, 1), lambda i: (i, 0)),
            pl.BlockSpec((BN, 1), lambda i: (i, 0)),
        ],
        out_shape=[
            jax.ShapeDtypeStruct((N, FW), jnp.float32),
            jax.ShapeDtypeStruct((N, 1), jnp.float32),
            jax.ShapeDtypeStruct((N, 1), jnp.float32),
            jax.ShapeDtypeStruct((N, 1), jnp.float32),
        ],
    )(acc, ws, haug, b, W, asrc, adst)


def _final_body(acc_ref, ws_ref, h_ref, b_ref, Wl_ref, bl_ref,
                batch_ref, sums_ref, cnts_ref, res_ref):
    i = pl.program_id(0)
    xn = _node_out(acc_ref[...], ws_ref[...], h_ref[...], b_ref[...])
    z = jnp.dot(xn, Wl_ref[...], preferred_element_type=jnp.float32,
                 precision=lax.Precision.HIGHEST)
    gid = lax.broadcasted_iota(jnp.int32, (1, G), 1)
    m = (batch_ref[...] == gid).astype(jnp.float32)
    part = jnp.sum(m * z, axis=0, keepdims=True)
    cnt = jnp.sum(m, axis=0, keepdims=True)

    @pl.when(i == 0)
    def _():
        sums_ref[...] = part
        cnts_ref[...] = cnt

    @pl.when(i > 0)
    def _():
        sums_ref[...] = sums_ref[...] + part
        cnts_ref[...] = cnts_ref[...] + cnt

    @pl.when(i == NB - 1)
    def _():
        res_ref[...] = (sums_ref[...] / jnp.maximum(cnts_ref[...], 1.0)
                        + bl_ref[...])


def _final(acc, ws, haug, b, W_lin, b_lin, batch2d):
    return pl.pallas_call(
        _final_body,
        grid=(NB,),
        in_specs=[
            pl.BlockSpec((2, BN, FW), lambda i: (0, i, 0)),
            pl.BlockSpec((BN, 1), lambda i: (i, 0)),
            pl.BlockSpec((BN, FW), lambda i: (i, 0)),
            pl.BlockSpec((1, F), lambda i: (0, 0)),
            pl.BlockSpec((F, 1), lambda i: (0, 0)),
            pl.BlockSpec((1, 1), lambda i: (0, 0)),
            pl.BlockSpec((BN, 1), lambda i: (i, 0)),
        ],
        out_specs=[
            pl.BlockSpec((1, G), lambda i: (0, 0)),
            pl.BlockSpec((1, G), lambda i: (0, 0)),
            pl.BlockSpec((1, G), lambda i: (0, 0)),
        ],
        out_shape=[
            jax.ShapeDtypeStruct((1, G), jnp.float32),
            jax.ShapeDtypeStruct((1, G), jnp.float32),
            jax.ShapeDtypeStruct((1, G), jnp.float32),
        ],
    )(acc, ws, haug, b, W_lin, b_lin, batch2d)


def _sc_mesh():
    return plsc.VectorSubcoreMesh(core_axis_name="c", subcore_axis_name="s",
                                  num_cores=NC, num_subcores=NS)


def _sc_w_body(as_hbm, ad_hbm, src_hbm, dst_hbm,
               w_hbm,
               src_t, dst_t, asv, adv, w_t):
    c = lax.axis_index("c")
    s = lax.axis_index("s")
    wid = s * NC + c
    lane = lax.iota(jnp.int32, 16)

    pltpu.sync_copy(src_hbm.at[pl.ds(wid * T, T)], src_t)
    pltpu.sync_copy(dst_hbm.at[pl.ds(wid * T, T)], dst_t)
    pltpu.sync_copy(as_hbm, asv)
    pltpu.sync_copy(ad_hbm, adv.at[pl.ds(0, N)])

    def _chunk(k, carry):
        base = k * C
        for v in range(C // 16):
            s16 = src_t[pl.ds(base + v * 16, 16)]
            d16 = dst_t[pl.ds(base + v * 16, 16)]
            al = (plsc.load_gather(asv, [s16])
                  + plsc.load_gather(adv, [d16]))
            al = jnp.maximum(al, 0.2 * al)
            w = jnp.exp(al)
            eid = wid * T + base + v * 16 + lane
            w = jnp.where(eid < E, w, 0.0)
            w_t[pl.ds(base + v * 16, 16)] = w
        return carry

    lax.fori_loop(0, NCH, _chunk, 0)
    pltpu.sync_copy(w_t, w_hbm.at[pl.ds(wid * T, T)])


def _sc_msg_body(h_hbm, w_hbm, src_hbm, dst_hbm,
                 acc_hbm,
                 src_g, dst_g, w_g,
                 srcbuf0, dstbuf0, wbuf0, rows0,
                 srcbuf1, dstbuf1, wbuf1, rows1,
                 acc_s, sg0, sg1, ss0, ss1):
    c = lax.axis_index("c")
    s = lax.axis_index("s")
    wid = s * NC + c
    zero16 = jnp.zeros((16,), jnp.float32)

    def _zr(rows):
        def _z(i, carry):
            for q in range(FW // 16):
                rows[i, pl.ds(q * 16, 16)] = zero16
            return carry
        lax.fori_loop(0, CB, _z, 0)

    _zr(rows0)
    _zr(rows1)
    for ss_ in range(NS):
        @pl.when(s == ss_)
        def _():
            for j in range(RPT // (2 * CB)):
                pltpu.sync_copy(rows0,
                                acc_s.at[pl.ds(ss_ * RPT + 2 * j * CB, CB)])
                pltpu.sync_copy(rows1,
                                acc_s.at[pl.ds(ss_ * RPT + (2 * j + 1) * CB,
                                               CB)])
    plsc.subcore_barrier()

    def _fill(base, srcbuf, dstbuf, wbuf):
        for v in range(CB // 16):
            srcbuf[pl.ds(v * 16, 16)] = src_g[pl.ds(base + v * 16, 16)]
            dstbuf[pl.ds(v * 16, 16)] = dst_g[pl.ds(base + v * 16, 16)]
            wbuf[pl.ds(v * 16, 16)] = w_g[pl.ds(base + v * 16, 16)]

    def _scale(rows, wbuf):
        def _sc(j, carry):
            wj = plsc.load_gather(
                wbuf, [jnp.broadcast_to(j, (16,)).astype(jnp.int32)])
            for q in range(FW // 16):
                rows[j, pl.ds(q * 16, 16)] = rows[j, pl.ds(q * 16, 16)] * wj
            return carry
        lax.fori_loop(0, CB, _sc, 0, unroll=2)

    def _seg(g, carry):
        off = wid * T + g * S
        pltpu.sync_copy(src_hbm.at[pl.ds(off, S)], src_g)
        pltpu.sync_copy(dst_hbm.at[pl.ds(off, S)], dst_g)
        pltpu.sync_copy(w_hbm.at[pl.ds(off, S)], w_g)

        def _pair(kk, carry2):
            base0 = kk * 2 * CB
            base1 = base0 + CB
            _fill(base0, srcbuf0, dstbuf0, wbuf0)
            g0 = pltpu.async_copy(h_hbm.at[srcbuf0], rows0, sg0)
            _fill(base1, srcbuf1, dstbuf1, wbuf1)
            g1 = pltpu.async_copy(h_hbm.at[srcbuf1], rows1, sg1)
            g0.wait()
            _scale(rows0, wbuf0)
            sc0 = pltpu.async_copy(rows0, acc_s.at[dstbuf0], ss0, add=True)
            g1.wait()
            _scale(rows1, wbuf1)
            sc1 = pltpu.async_copy(rows1, acc_s.at[dstbuf1], ss1, add=True)
            sc0.wait()
            sc1.wait()
            return carry2

        lax.fori_loop(0, PAIRS, _pair, 0)
        return carry

    lax.fori_loop(0, NSEG, _seg, 0)
    plsc.subcore_barrier()
    for ss_ in range(NS):
        @pl.when(s == ss_)
        def _():
            pltpu.sync_copy(acc_s.at[pl.ds(ss_ * RPT, RPT)],
                            acc_hbm.at[c, pl.ds(ss_ * RPT, RPT)])


def _edge_pass(haug, a_s, a_d, src_pad, dst_pad):
    w_pad = pl.kernel(
        _sc_w_body,
        out_type=jax.ShapeDtypeStruct((E_PAD,), jnp.float32),
        mesh=_sc_mesh(),
        compiler_params=_SC_PARAMS,
        scratch_types=[
            pltpu.VMEM((T,), jnp.int32),
            pltpu.VMEM((T,), jnp.int32),
            pltpu.VMEM((N,), jnp.float32),
            pltpu.VMEM((NPAD,), jnp.float32),
            pltpu.VMEM((T,), jnp.float32),
        ],
    )(a_s, a_d, src_pad, dst_pad)
    acc = pl.kernel(
        _sc_msg_body,
        out_type=jax.ShapeDtypeStruct((NC, NPAD, FW), jnp.float32),
        mesh=_sc_mesh(),
        compiler_params=_SC_PARAMS,
        scratch_types=[
            pltpu.VMEM((S,), jnp.int32),
            pltpu.VMEM((S,), jnp.int32),
            pltpu.VMEM((S,), jnp.float32),
            pltpu.VMEM((CB,), jnp.int32),
            pltpu.VMEM((CB,), jnp.int32),
            pltpu.VMEM((CB,), jnp.float32),
            pltpu.VMEM((CB, FW), jnp.float32),
            pltpu.VMEM((CB,), jnp.int32),
            pltpu.VMEM((CB,), jnp.int32),
            pltpu.VMEM((CB,), jnp.float32),
            pltpu.VMEM((CB, FW), jnp.float32),
            pltpu.VMEM_SHARED((NPAD, FW), jnp.float32),
            pltpu.SemaphoreType.DMA,
            pltpu.SemaphoreType.DMA,
            pltpu.SemaphoreType.DMA,
            pltpu.SemaphoreType.DMA,
        ],
    )(haug, w_pad, src_pad, dst_pad)
    return acc


def kernel(x, edge_index, batch,
           W0, asrc0, adst0, b0,
           W1, asrc1, adst1, b1,
           W2, asrc2, adst2, b2,
           W_lin, b_lin):
    pad = jnp.zeros((E_PAD - E,), jnp.int32)
    src_pad = jnp.concatenate([edge_index[0], pad])
    padd = N + (jnp.arange(E_PAD - E, dtype=jnp.int32) % (NPAD - N))
    dst_pad = jnp.concatenate([edge_index[1], padd])

    h, a_s, a_d, ws = _layer_in(x, W0, asrc0, adst0)
    acc = _edge_pass(h, a_s.reshape(N), a_d.reshape(N), src_pad, dst_pad)
    h, a_s, a_d, ws = _combine(acc, ws, h, b0.reshape(1, F), W1, asrc1, adst1)
    acc = _edge_pass(h, a_s.reshape(N), a_d.reshape(N), src_pad, dst_pad)
    h, a_s, a_d, ws = _combine(acc, ws, h, b1.reshape(1, F), W2, asrc2, adst2)
    acc = _edge_pass(h, a_s.reshape(N), a_d.reshape(N), src_pad, dst_pad)
    _, _, res = _final(acc, ws, h, b2.reshape(1, F), W_lin,
                       b_lin.reshape(1, 1), batch.reshape(N, 1))
    return res.reshape(G)
